# Initial kernel scaffold; baseline (speedup 1.0000x reference)
import jax
import jax.numpy as jnp
from jax.experimental import pallas as pl


def kernel(x, edge_index, W_gcn, gamma_bn, beta_bn, W_film, b_film):
    raise NotImplementedError("write your pallas kernel here")


# speedup vs baseline: 12.3932x; 12.3932x over previous
import functools

import jax
import jax.numpy as jnp
from jax import lax
from jax.experimental import pallas as pl
from jax.experimental.pallas import tpu as pltpu
from jax.experimental.pallas import tpu_sc as plsc

N = 10000
D = 128
E = 320000
BN_EPS = 1e-4

NC, NS, L = 2, 16, 16
DH = D // NC
CH = 128
NCH = 160
EPAD = NS * NCH * CH
NPAD = 10112
RPS = NPAD // NS

@functools.lru_cache(maxsize=None)
def _sc_kernels():
    mesh = plsc.VectorSubcoreMesh(core_axis_name="c", subcore_axis_name="s")
    cp = pltpu.CompilerParams(use_tc_tiling_on_sc=False)

    @functools.partial(
        pl.kernel,
        out_type=jax.ShapeDtypeStruct((NC, NPAD, L), jnp.float32),
        mesh=mesh,
        scratch_types=[
            pltpu.VMEM((NCH, CH), jnp.int32),
            pltpu.VMEM((CH, L), jnp.float32),
            pltpu.VMEM((RPS, L), jnp.float32),
            pltpu.VMEM_SHARED((NPAD, L), jnp.float32),
        ],
        compiler_params=cp,
    )
    def _degree_kernel(idx_hbm, out_hbm, idx_v, ones_v, zer_v, acc):
        c = lax.axis_index("c")
        s = lax.axis_index("s")

        @pl.loop(0, CH)
        def _(i):
            ones_v[i, :] = jnp.full((L,), 1.0, jnp.float32)

        @pl.loop(0, RPS)
        def _(i):
            zer_v[i, :] = jnp.zeros((L,), jnp.float32)

        pltpu.sync_copy(zer_v, acc.at[pl.ds(s * RPS, RPS)])
        pltpu.sync_copy(idx_hbm.at[c, s], idx_v)
        plsc.subcore_barrier()

        @pl.loop(0, NCH)
        def _(j):
            pltpu.sync_copy(ones_v, acc.at[idx_v.at[j]], add=True)

        plsc.subcore_barrier()
        pltpu.sync_copy(acc.at[pl.ds(s * RPS, RPS)],
                        out_hbm.at[c].at[pl.ds(s * RPS, RPS)])

    @functools.partial(
        pl.kernel,
        out_type=jax.ShapeDtypeStruct((NC, NPAD, DH), jnp.float32),
        mesh=mesh,
        scratch_types=[
            pltpu.VMEM((NCH, CH), jnp.int32),
            pltpu.VMEM((NCH, CH), jnp.int32),
            pltpu.VMEM((CH, DH), jnp.float32),
            pltpu.VMEM((CH, DH), jnp.float32),
            pltpu.VMEM_SHARED((NPAD, DH), jnp.float32),
            pltpu.SemaphoreType.DMA,
            pltpu.SemaphoreType.DMA,
        ],
        compiler_params=cp,
    )
    def _edge_kernel(xs_hbm, src_hbm, dst_hbm, out_hbm,
                     sidx, didx, rows0, rows1, acc, sem0, sem1):
        c = lax.axis_index("c")
        s = lax.axis_index("s")

        @pl.loop(0, CH)
        def _(i):
            @pl.loop(0, DH, step=L)
            def _(k):
                rows0[i, pl.ds(k, L)] = jnp.zeros((L,), jnp.float32)

        @pl.loop(0, RPS // CH)
        def _(b):
            pltpu.sync_copy(rows0, acc.at[pl.ds(s * RPS + b * CH, CH)])
        pltpu.sync_copy(rows0.at[pl.ds(0, RPS % CH)],
                        acc.at[pl.ds(s * RPS + (RPS // CH) * CH, RPS % CH)])

        pltpu.sync_copy(src_hbm.at[s], sidx)
        pltpu.sync_copy(dst_hbm.at[s], didx)
        plsc.subcore_barrier()

        xs_half = xs_hbm.at[c]

        @pl.loop(0, NCH // 2)
        def _(p):
            j = p * 2
            g0 = pltpu.async_copy(xs_half.at[sidx.at[j]], rows0, sem0)
            g1 = pltpu.async_copy(xs_half.at[sidx.at[j + 1]], rows1, sem1)
            g0.wait()
            pltpu.sync_copy(rows0, acc.at[didx.at[j]], add=True)
            g1.wait()
            pltpu.sync_copy(rows1, acc.at[didx.at[j + 1]], add=True)

        plsc.subcore_barrier()
        pltpu.sync_copy(acc.at[pl.ds(s * RPS, RPS)],
                        out_hbm.at[c].at[pl.ds(s * RPS, RPS)])

    return _degree_kernel, _edge_kernel


def _scale_body(x_ref, deg_ref, o_ref):
    deg = jnp.maximum(deg_ref[0][:N, 0], 1.0)
    xs = x_ref[...] * lax.rsqrt(deg)[:, None]
    o_ref[0, :N, :] = xs[:, :DH]
    o_ref[1, :N, :] = xs[:, DH:]
    o_ref[:, N:, :] = jnp.zeros((NC, NPAD - N, DH), jnp.float32)


def _dense_body(p_ref, deg_ref, x_ref, wg_ref, g_ref, b_ref, wf_ref, bf_ref,
                o_ref):
    p = jnp.concatenate([p_ref[0][:N, :], p_ref[1][:N, :]], axis=-1)
    deg_in = jnp.maximum(deg_ref[1][:N, 0], 1.0)
    agg = p * lax.rsqrt(deg_in)[:, None]
    gcn = jnp.dot(agg, wg_ref[...], precision=lax.Precision.HIGHEST)
    mean = jnp.mean(gcn, axis=0)
    var = jnp.mean((gcn - mean) ** 2, axis=0)
    yblo = g_ref[...] * (gcn - mean) * lax.rsqrt(var + BN_EPS) + b_ref[...]
    film = jnp.dot(yblo, wf_ref[...], precision=lax.Precision.HIGHEST)
    film = film + bf_ref[...]
    z = film[:, :D] * yblo + film[:, D:]
    o_ref[...] = jnp.maximum(z, 0.0) + x_ref[...]


_scale_call = pl.pallas_call(
    _scale_body,
    out_shape=jax.ShapeDtypeStruct((NC, NPAD, DH), jnp.float32),
)

_dense_call = pl.pallas_call(
    _dense_body,
    out_shape=jax.ShapeDtypeStruct((N, D), jnp.float32),
    compiler_params=pltpu.CompilerParams(vmem_limit_bytes=64 * 1024 * 1024),
)


def kernel(x, edge_index, W_gcn, gamma_bn, beta_bn, W_film, b_film):
    ei = edge_index.astype(jnp.int32)
    pad = jnp.full((EPAD - E,), N, jnp.int32)
    srcp = jnp.concatenate([ei[0], pad])
    dstp = jnp.concatenate([ei[1], pad])
    idx1 = jnp.stack([srcp, dstp]).reshape(NC, NS, NCH, CH)
    src3 = srcp.reshape(NS, NCH, CH)
    dst3 = dstp.reshape(NS, NCH, CH)

    degree_kernel, edge_kernel = _sc_kernels()
    degp = degree_kernel(idx1)
    xs = _scale_call(x, degp)
    partials = edge_kernel(xs, src3, dst3)
    return _dense_call(partials, degp, x, W_gcn, gamma_bn, beta_bn,
                       W_film, b_film)


# speedup vs baseline: 13.1882x; 1.0641x over previous
import functools

import jax
import jax.numpy as jnp
from jax import lax
from jax.experimental import pallas as pl
from jax.experimental.pallas import tpu as pltpu
from jax.experimental.pallas import tpu_sc as plsc

N = 10000
D = 128
E = 320000
BN_EPS = 1e-4

NC, NS, L = 2, 16, 16
DH = D // NC
CH = 128
NCH = 160
EPAD = NS * NCH * CH
NPAD = 10112
RPS = NPAD // NS

@functools.lru_cache(maxsize=None)
def _sc_kernels():
    mesh = plsc.VectorSubcoreMesh(core_axis_name="c", subcore_axis_name="s")
    cp = pltpu.CompilerParams(use_tc_tiling_on_sc=False)

    @functools.partial(
        pl.kernel,
        out_type=jax.ShapeDtypeStruct((NC, NPAD, L), jnp.float32),
        mesh=mesh,
        scratch_types=[
            pltpu.VMEM((NCH, CH), jnp.int32),
            pltpu.VMEM((CH, L), jnp.float32),
            pltpu.VMEM((RPS, L), jnp.float32),
            pltpu.VMEM_SHARED((NPAD, L), jnp.float32),
        ],
        compiler_params=cp,
    )
    def _degree_kernel(idx_hbm, out_hbm, idx_v, ones_v, zer_v, acc):
        c = lax.axis_index("c")
        s = lax.axis_index("s")

        @pl.loop(0, CH)
        def _(i):
            ones_v[i, :] = jnp.full((L,), 1.0, jnp.float32)

        @pl.loop(0, RPS)
        def _(i):
            zer_v[i, :] = jnp.zeros((L,), jnp.float32)

        pltpu.sync_copy(zer_v, acc.at[pl.ds(s * RPS, RPS)])
        pltpu.sync_copy(idx_hbm.at[c, s], idx_v)
        plsc.subcore_barrier()

        @pl.loop(0, NCH)
        def _(j):
            pltpu.sync_copy(ones_v, acc.at[idx_v.at[j]], add=True)

        plsc.subcore_barrier()
        pltpu.sync_copy(acc.at[pl.ds(s * RPS, RPS)],
                        out_hbm.at[c].at[pl.ds(s * RPS, RPS)])

    @functools.partial(
        pl.kernel,
        out_type=jax.ShapeDtypeStruct((NC, NPAD, DH), jnp.float32),
        mesh=mesh,
        scratch_types=[
            pltpu.VMEM((NCH, CH), jnp.int32),
            pltpu.VMEM((NCH, CH), jnp.int32),
            pltpu.VMEM((4, CH, DH), jnp.float32),
            pltpu.VMEM_SHARED((NPAD, DH), jnp.float32),
            [pltpu.SemaphoreType.DMA] * 4,
            [pltpu.SemaphoreType.DMA] * 4,
        ],
        compiler_params=cp,
    )
    def _edge_kernel(xs_hbm, src_hbm, dst_hbm, out_hbm,
                     sidx, didx, rows, acc, gsem, ssem):
        c = lax.axis_index("c")
        s = lax.axis_index("s")

        @pl.loop(0, CH)
        def _(i):
            @pl.loop(0, DH, step=L)
            def _(k):
                rows[0, i, pl.ds(k, L)] = jnp.zeros((L,), jnp.float32)

        zbuf = rows.at[0]

        @pl.loop(0, RPS // CH)
        def _(b):
            pltpu.sync_copy(zbuf, acc.at[pl.ds(s * RPS + b * CH, CH)])
        pltpu.sync_copy(zbuf.at[pl.ds(0, RPS % CH)],
                        acc.at[pl.ds(s * RPS + (RPS // CH) * CH, RPS % CH)])

        pltpu.sync_copy(src_hbm.at[s], sidx)
        pltpu.sync_copy(dst_hbm.at[s], didx)
        plsc.subcore_barrier()

        xs_half = xs_hbm.at[c]

        @pl.loop(0, NCH // 4)
        def _(q):
            j = q * 4
            g0 = pltpu.async_copy(xs_half.at[sidx.at[j]], rows.at[0], gsem[0])
            g1 = pltpu.async_copy(xs_half.at[sidx.at[j + 1]], rows.at[1],
                                  gsem[1])
            g0.wait()
            s0 = pltpu.async_copy(rows.at[0], acc.at[didx.at[j]], ssem[0],
                                  add=True)
            g2 = pltpu.async_copy(xs_half.at[sidx.at[j + 2]], rows.at[2],
                                  gsem[2])
            g1.wait()
            s1 = pltpu.async_copy(rows.at[1], acc.at[didx.at[j + 1]], ssem[1],
                                  add=True)
            g3 = pltpu.async_copy(xs_half.at[sidx.at[j + 3]], rows.at[3],
                                  gsem[3])
            g2.wait()
            s2 = pltpu.async_copy(rows.at[2], acc.at[didx.at[j + 2]], ssem[2],
                                  add=True)
            g3.wait()
            s3 = pltpu.async_copy(rows.at[3], acc.at[didx.at[j + 3]], ssem[3],
                                  add=True)
            s0.wait()
            s1.wait()
            s2.wait()
            s3.wait()

        plsc.subcore_barrier()
        pltpu.sync_copy(acc.at[pl.ds(s * RPS, RPS)],
                        out_hbm.at[c].at[pl.ds(s * RPS, RPS)])

    return _degree_kernel, _edge_kernel


def _scale_body(x_ref, deg_ref, o_ref):
    deg = jnp.maximum(deg_ref[0][:N, 0], 1.0)
    xs = x_ref[...] * lax.rsqrt(deg)[:, None]
    o_ref[0, :N, :] = xs[:, :DH]
    o_ref[1, :N, :] = xs[:, DH:]
    o_ref[:, N:, :] = jnp.zeros((NC, NPAD - N, DH), jnp.float32)


def _dense_body(p_ref, deg_ref, x_ref, wg_ref, g_ref, b_ref, wf_ref, bf_ref,
                o_ref):
    p = jnp.concatenate([p_ref[0][:N, :], p_ref[1][:N, :]], axis=-1)
    deg_in = jnp.maximum(deg_ref[1][:N, 0], 1.0)
    agg = p * lax.rsqrt(deg_in)[:, None]
    gcn = jnp.dot(agg, wg_ref[...], precision=lax.Precision.HIGHEST)
    mean = jnp.mean(gcn, axis=0)
    var = jnp.mean((gcn - mean) ** 2, axis=0)
    yblo = g_ref[...] * (gcn - mean) * lax.rsqrt(var + BN_EPS) + b_ref[...]
    film = jnp.dot(yblo, wf_ref[...], precision=lax.Precision.HIGHEST)
    film = film + bf_ref[...]
    z = film[:, :D] * yblo + film[:, D:]
    o_ref[...] = jnp.maximum(z, 0.0) + x_ref[...]


_scale_call = pl.pallas_call(
    _scale_body,
    out_shape=jax.ShapeDtypeStruct((NC, NPAD, DH), jnp.float32),
)

_dense_call = pl.pallas_call(
    _dense_body,
    out_shape=jax.ShapeDtypeStruct((N, D), jnp.float32),
    compiler_params=pltpu.CompilerParams(vmem_limit_bytes=64 * 1024 * 1024),
)


def kernel(x, edge_index, W_gcn, gamma_bn, beta_bn, W_film, b_film):
    ei = edge_index.astype(jnp.int32)
    pad = jnp.full((EPAD - E,), N, jnp.int32)
    srcp = jnp.concatenate([ei[0], pad])
    dstp = jnp.concatenate([ei[1], pad])
    idx1 = jnp.stack([srcp, dstp]).reshape(NC, NS, NCH, CH)
    src3 = srcp.reshape(NS, NCH, CH)
    dst3 = dstp.reshape(NS, NCH, CH)

    degree_kernel, edge_kernel = _sc_kernels()
    degp = degree_kernel(idx1)
    xs = _scale_call(x, degp)
    partials = edge_kernel(xs, src3, dst3)
    return _dense_call(partials, degp, x, W_gcn, gamma_bn, beta_bn,
                       W_film, b_film)


# speedup vs baseline: 13.2481x; 1.0045x over previous
import functools

import jax
import jax.numpy as jnp
from jax import lax
from jax.experimental import pallas as pl
from jax.experimental.pallas import tpu as pltpu
from jax.experimental.pallas import tpu_sc as plsc

N = 10000
D = 128
E = 320000
BN_EPS = 1e-4

NC, NS, L = 2, 16, 16
DH = D // NC
CH = 128
KR = 1
CHUNK = KR * CH
NCH = 160
KR1 = 4
NCH1 = 40
EPAD = NS * NCH * CHUNK
NPAD = 10112
RPS = NPAD // NS

@functools.lru_cache(maxsize=None)
def _sc_kernels():
    mesh = plsc.VectorSubcoreMesh(core_axis_name="c", subcore_axis_name="s")
    cp = pltpu.CompilerParams(use_tc_tiling_on_sc=False)

    @functools.partial(
        pl.kernel,
        out_type=jax.ShapeDtypeStruct((NC, NPAD, L), jnp.float32),
        mesh=mesh,
        scratch_types=[
            pltpu.VMEM((NCH1 * KR1, CH), jnp.int32),
            pltpu.VMEM((CH, L), jnp.float32),
            pltpu.VMEM((RPS, L), jnp.float32),
            pltpu.VMEM_SHARED((NPAD, L), jnp.float32),
            [pltpu.SemaphoreType.DMA] * 4,
        ],
        compiler_params=cp,
    )
    def _degree_kernel(idx_hbm, out_hbm, idx_v, ones_v, zer_v, acc, dsem):
        c = lax.axis_index("c")
        s = lax.axis_index("s")

        @pl.loop(0, CH)
        def _(i):
            ones_v[i, :] = jnp.full((L,), 1.0, jnp.float32)

        @pl.loop(0, RPS)
        def _(i):
            zer_v[i, :] = jnp.zeros((L,), jnp.float32)

        pltpu.sync_copy(zer_v, acc.at[pl.ds(s * RPS, RPS)])
        pltpu.sync_copy(idx_hbm.at[c, s], idx_v)
        plsc.subcore_barrier()

        @pl.loop(0, NCH1 * KR1 // 4)
        def _(q):
            j = q * 4
            d0 = pltpu.async_copy(ones_v, acc.at[idx_v.at[j]], dsem[0],
                                  add=True)
            d1 = pltpu.async_copy(ones_v, acc.at[idx_v.at[j + 1]], dsem[1],
                                  add=True)
            d2 = pltpu.async_copy(ones_v, acc.at[idx_v.at[j + 2]], dsem[2],
                                  add=True)
            d3 = pltpu.async_copy(ones_v, acc.at[idx_v.at[j + 3]], dsem[3],
                                  add=True)
            d0.wait()
            d1.wait()
            d2.wait()
            d3.wait()

        plsc.subcore_barrier()
        pltpu.sync_copy(acc.at[pl.ds(s * RPS, RPS)],
                        out_hbm.at[c].at[pl.ds(s * RPS, RPS)])

    @functools.partial(
        pl.kernel,
        out_type=jax.ShapeDtypeStruct((NC, NPAD, DH), jnp.float32),
        mesh=mesh,
        scratch_types=[
            pltpu.VMEM((NCH, CHUNK), jnp.int32),
            pltpu.VMEM((NCH, CHUNK), jnp.int32),
            pltpu.VMEM((4, CHUNK, DH), jnp.float32),
            pltpu.VMEM_SHARED((NPAD, DH), jnp.float32),
            [pltpu.SemaphoreType.DMA] * 4,
            [pltpu.SemaphoreType.DMA] * 4,
        ],
        compiler_params=cp,
    )
    def _edge_kernel(xs_hbm, src_hbm, dst_hbm, out_hbm,
                     sidx, didx, rows, acc, gsem, ssem):
        c = lax.axis_index("c")
        s = lax.axis_index("s")

        @pl.loop(0, CHUNK)
        def _(i):
            @pl.loop(0, DH, step=L)
            def _(k):
                rows[0, i, pl.ds(k, L)] = jnp.zeros((L,), jnp.float32)

        zbuf = rows.at[0]

        @pl.loop(0, RPS // CHUNK)
        def _(b):
            pltpu.sync_copy(zbuf, acc.at[pl.ds(s * RPS + b * CHUNK, CHUNK)])
        pltpu.sync_copy(zbuf.at[pl.ds(0, RPS % CHUNK)],
                        acc.at[pl.ds(s * RPS + (RPS // CHUNK) * CHUNK,
                                     RPS % CHUNK)])

        pltpu.sync_copy(src_hbm.at[s], sidx)
        pltpu.sync_copy(dst_hbm.at[s], didx)
        plsc.subcore_barrier()

        xs_half = xs_hbm.at[c]

        def _scat(t, j, sem):
            return pltpu.async_copy(rows.at[t], acc.at[didx.at[j]], sem,
                                    add=True)

        @pl.loop(0, NCH // 4)
        def _(q):
            j = q * 4
            g0 = pltpu.async_copy(xs_half.at[sidx.at[j]], rows.at[0], gsem[0])
            g1 = pltpu.async_copy(xs_half.at[sidx.at[j + 1]], rows.at[1],
                                  gsem[1])
            g0.wait()
            s0 = _scat(0, j, ssem[0])
            g2 = pltpu.async_copy(xs_half.at[sidx.at[j + 2]], rows.at[2],
                                  gsem[2])
            g1.wait()
            s1 = _scat(1, j + 1, ssem[1])
            g3 = pltpu.async_copy(xs_half.at[sidx.at[j + 3]], rows.at[3],
                                  gsem[3])
            g2.wait()
            s2 = _scat(2, j + 2, ssem[2])
            g3.wait()
            s3 = _scat(3, j + 3, ssem[3])
            s0.wait()
            s1.wait()
            s2.wait()
            s3.wait()

        plsc.subcore_barrier()
        pltpu.sync_copy(acc.at[pl.ds(s * RPS, RPS)],
                        out_hbm.at[c].at[pl.ds(s * RPS, RPS)])

    return _degree_kernel, _edge_kernel


def _scale_body(x_ref, deg_ref, o_ref):
    deg = jnp.maximum(deg_ref[0][:N, 0], 1.0)
    xs = x_ref[...] * lax.rsqrt(deg)[:, None]
    o_ref[0, :N, :] = xs[:, :DH]
    o_ref[1, :N, :] = xs[:, DH:]
    o_ref[:, N:, :] = jnp.zeros((NC, NPAD - N, DH), jnp.float32)


def _dense_body(p_ref, deg_ref, x_ref, wg_ref, g_ref, b_ref, wf_ref, bf_ref,
                o_ref):
    p = jnp.concatenate([p_ref[0][:N, :], p_ref[1][:N, :]], axis=-1)
    deg_in = jnp.maximum(deg_ref[1][:N, 0], 1.0)
    agg = p * lax.rsqrt(deg_in)[:, None]
    gcn = jnp.dot(agg, wg_ref[...], precision=lax.Precision.HIGHEST)
    mean = jnp.mean(gcn, axis=0)
    var = jnp.mean((gcn - mean) ** 2, axis=0)
    yblo = g_ref[...] * (gcn - mean) * lax.rsqrt(var + BN_EPS) + b_ref[...]
    film = jnp.dot(yblo, wf_ref[...], precision=lax.Precision.HIGHEST)
    film = film + bf_ref[...]
    z = film[:, :D] * yblo + film[:, D:]
    o_ref[...] = jnp.maximum(z, 0.0) + x_ref[...]


_scale_call = pl.pallas_call(
    _scale_body,
    out_shape=jax.ShapeDtypeStruct((NC, NPAD, DH), jnp.float32),
)

_dense_call = pl.pallas_call(
    _dense_body,
    out_shape=jax.ShapeDtypeStruct((N, D), jnp.float32),
    compiler_params=pltpu.CompilerParams(vmem_limit_bytes=64 * 1024 * 1024),
)


def kernel(x, edge_index, W_gcn, gamma_bn, beta_bn, W_film, b_film):
    ei = edge_index.astype(jnp.int32)
    pad = jnp.full((EPAD - E,), N, jnp.int32)
    srcp = jnp.concatenate([ei[0], pad])
    dstp = jnp.concatenate([ei[1], pad])
    idx1 = jnp.stack([srcp, dstp]).reshape(NC, NS, NCH1 * KR1, CH)
    src3 = srcp.reshape(NS, NCH, CHUNK)
    dst3 = dstp.reshape(NS, NCH, CHUNK)

    degree_kernel, edge_kernel = _sc_kernels()
    degp = degree_kernel(idx1)
    xs = _scale_call(x, degp)
    partials = edge_kernel(xs, src3, dst3)
    return _dense_call(partials, degp, x, W_gcn, gamma_bn, beta_bn,
                       W_film, b_film)


# speedup vs baseline: 28.0355x; 2.1162x over previous
import functools

import jax
import jax.numpy as jnp
from jax import lax
from jax.experimental import pallas as pl
from jax.experimental.pallas import tpu as pltpu
from jax.experimental.pallas import tpu_sc as plsc

N = 10000
D = 128
E = 320000
BN_EPS = 1e-4

NC, NS, L = 2, 16, 16
DH = D // NC
CH = 128
KR = 1
CHUNK = KR * CH
NCH = 160
KR1 = 4
NCH1 = 40
EPAD = NS * NCH * CHUNK
NPAD = 10112
RPS = NPAD // NS

@functools.lru_cache(maxsize=None)
def _sc_kernels():
    mesh = plsc.VectorSubcoreMesh(core_axis_name="c", subcore_axis_name="s")
    cp = pltpu.CompilerParams(use_tc_tiling_on_sc=False)

    @functools.partial(
        pl.kernel,
        out_type=jax.ShapeDtypeStruct((NC, NPAD, L), jnp.float32),
        mesh=mesh,
        scratch_types=[
            pltpu.VMEM((NCH1 * KR1, CH), jnp.int32),
            pltpu.VMEM((CH, L), jnp.float32),
            pltpu.VMEM((RPS, L), jnp.float32),
            pltpu.VMEM_SHARED((NPAD, L), jnp.float32),
            [pltpu.SemaphoreType.DMA] * 4,
        ],
        compiler_params=cp,
    )
    def _degree_kernel(idx_hbm, out_hbm, idx_v, ones_v, zer_v, acc, dsem):
        c = lax.axis_index("c")
        s = lax.axis_index("s")

        @pl.loop(0, CH)
        def _(i):
            ones_v[i, :] = jnp.full((L,), 1.0, jnp.float32)

        @pl.loop(0, RPS)
        def _(i):
            zer_v[i, :] = jnp.zeros((L,), jnp.float32)

        pltpu.sync_copy(zer_v, acc.at[pl.ds(s * RPS, RPS)])
        pltpu.sync_copy(idx_hbm.at[c, s], idx_v)
        plsc.subcore_barrier()

        @pl.loop(0, NCH1 * KR1 // 4)
        def _(q):
            j = q * 4
            d0 = pltpu.async_copy(ones_v, acc.at[idx_v.at[j]], dsem[0],
                                  add=True)
            d1 = pltpu.async_copy(ones_v, acc.at[idx_v.at[j + 1]], dsem[1],
                                  add=True)
            d2 = pltpu.async_copy(ones_v, acc.at[idx_v.at[j + 2]], dsem[2],
                                  add=True)
            d3 = pltpu.async_copy(ones_v, acc.at[idx_v.at[j + 3]], dsem[3],
                                  add=True)
            d0.wait()
            d1.wait()
            d2.wait()
            d3.wait()

        plsc.subcore_barrier()
        pltpu.sync_copy(acc.at[pl.ds(s * RPS, RPS)],
                        out_hbm.at[c].at[pl.ds(s * RPS, RPS)])

    @functools.partial(
        pl.kernel,
        out_type=jax.ShapeDtypeStruct((NC, NPAD, DH), jnp.bfloat16),
        mesh=mesh,
        scratch_types=[
            pltpu.VMEM((NCH, CHUNK), jnp.int32),
            pltpu.VMEM((NCH, CHUNK), jnp.int32),
            pltpu.VMEM((4, CHUNK, DH), jnp.bfloat16),
            pltpu.VMEM_SHARED((NPAD, DH), jnp.bfloat16),
            pltpu.VMEM_SHARED((NPAD, DH), jnp.bfloat16),
            [pltpu.SemaphoreType.DMA] * 4,
            [pltpu.SemaphoreType.DMA] * 4,
        ],
        compiler_params=cp,
    )
    def _edge_kernel(xs_hbm, src_hbm, dst_hbm, out_hbm,
                     sidx, didx, rows, acc, tbl, gsem, ssem):
        c = lax.axis_index("c")
        s = lax.axis_index("s")

        tstage = pltpu.async_copy(xs_hbm.at[c].at[pl.ds(s * RPS, RPS)],
                                  tbl.at[pl.ds(s * RPS, RPS)], ssem[0])

        @pl.loop(0, CHUNK)
        def _(i):
            @pl.loop(0, DH, step=2 * L)
            def _(k):
                rows[0, i, pl.ds(k, 2 * L)] = jnp.zeros((2 * L,),
                                                        jnp.bfloat16)

        zbuf = rows.at[0]

        @pl.loop(0, RPS // CHUNK)
        def _(b):
            pltpu.sync_copy(zbuf, acc.at[pl.ds(s * RPS + b * CHUNK, CHUNK)])
        pltpu.sync_copy(zbuf.at[pl.ds(0, RPS % CHUNK)],
                        acc.at[pl.ds(s * RPS + (RPS // CHUNK) * CHUNK,
                                     RPS % CHUNK)])

        pltpu.sync_copy(src_hbm.at[s], sidx)
        pltpu.sync_copy(dst_hbm.at[s], didx)
        tstage.wait()
        plsc.subcore_barrier()

        xs_half = tbl

        def _scat(t, j, sem):
            return pltpu.async_copy(rows.at[t], acc.at[didx.at[j]], sem,
                                    add=True)

        @pl.loop(0, NCH // 4)
        def _(q):
            j = q * 4
            g0 = pltpu.async_copy(xs_half.at[sidx.at[j]], rows.at[0], gsem[0])
            g1 = pltpu.async_copy(xs_half.at[sidx.at[j + 1]], rows.at[1],
                                  gsem[1])
            g0.wait()
            s0 = _scat(0, j, ssem[0])
            g2 = pltpu.async_copy(xs_half.at[sidx.at[j + 2]], rows.at[2],
                                  gsem[2])
            g1.wait()
            s1 = _scat(1, j + 1, ssem[1])
            g3 = pltpu.async_copy(xs_half.at[sidx.at[j + 3]], rows.at[3],
                                  gsem[3])
            g2.wait()
            s2 = _scat(2, j + 2, ssem[2])
            g3.wait()
            s3 = _scat(3, j + 3, ssem[3])
            s0.wait()
            s1.wait()
            s2.wait()
            s3.wait()

        plsc.subcore_barrier()
        pltpu.sync_copy(acc.at[pl.ds(s * RPS, RPS)],
                        out_hbm.at[c].at[pl.ds(s * RPS, RPS)])

    return _degree_kernel, _edge_kernel


def _scale_body(x_ref, deg_ref, o_ref):
    deg = jnp.maximum(deg_ref[0][:N, 0], 1.0)
    xs = (x_ref[...] * lax.rsqrt(deg)[:, None]).astype(jnp.bfloat16)
    o_ref[0, :N, :] = xs[:, :DH]
    o_ref[1, :N, :] = xs[:, DH:]
    o_ref[:, N:, :] = jnp.zeros((NC, NPAD - N, DH), jnp.bfloat16)


def _dense_body(p_ref, deg_ref, x_ref, wg_ref, g_ref, b_ref, wf_ref, bf_ref,
                o_ref):
    p = jnp.concatenate([p_ref[0][:N, :], p_ref[1][:N, :]],
                        axis=-1).astype(jnp.float32)
    deg_in = jnp.maximum(deg_ref[1][:N, 0], 1.0)
    agg = p * lax.rsqrt(deg_in)[:, None]
    gcn = jnp.dot(agg, wg_ref[...], precision=lax.Precision.HIGHEST)
    mean = jnp.mean(gcn, axis=0)
    var = jnp.mean((gcn - mean) ** 2, axis=0)
    yblo = g_ref[...] * (gcn - mean) * lax.rsqrt(var + BN_EPS) + b_ref[...]
    film = jnp.dot(yblo, wf_ref[...], precision=lax.Precision.HIGHEST)
    film = film + bf_ref[...]
    z = film[:, :D] * yblo + film[:, D:]
    o_ref[...] = jnp.maximum(z, 0.0) + x_ref[...]


_scale_call = pl.pallas_call(
    _scale_body,
    out_shape=jax.ShapeDtypeStruct((NC, NPAD, DH), jnp.bfloat16),
)

_dense_call = pl.pallas_call(
    _dense_body,
    out_shape=jax.ShapeDtypeStruct((N, D), jnp.float32),
    compiler_params=pltpu.CompilerParams(vmem_limit_bytes=64 * 1024 * 1024),
)


def kernel(x, edge_index, W_gcn, gamma_bn, beta_bn, W_film, b_film):
    ei = edge_index.astype(jnp.int32)
    pad = jnp.full((EPAD - E,), N, jnp.int32)
    srcp = jnp.concatenate([ei[0], pad])
    dstp = jnp.concatenate([ei[1], pad])
    idx1 = jnp.stack([srcp, dstp]).reshape(NC, NS, NCH1 * KR1, CH)
    src3 = srcp.reshape(NS, NCH, CHUNK)
    dst3 = dstp.reshape(NS, NCH, CHUNK)

    degree_kernel, edge_kernel = _sc_kernels()
    degp = degree_kernel(idx1)
    xs = _scale_call(x, degp)
    partials = edge_kernel(xs, src3, dst3)
    return _dense_call(partials, degp, x, W_gcn, gamma_bn, beta_bn,
                       W_film, b_film)
